# Initial kernel scaffold; baseline (speedup 1.0000x reference)
import jax
import jax.numpy as jnp
from jax.experimental import pallas as pl


def kernel(x, edge_index, batch, W_in_s, W_in_v, W_ss, W_vv, W_sv, W_vs, W_out):
    raise NotImplementedError("write your pallas kernel here")


# speedup vs baseline: 72.9715x; 72.9715x over previous
import functools

import jax
import jax.numpy as jnp
from jax import lax
from jax.experimental import pallas as pl
from jax.experimental.pallas import tpu as pltpu
from jax.experimental.pallas import tpu_sc as plsc

NN = 50000
EE = 800000
NB = 64
NC = 2
NSUB = 16
NW = NC * NSUB

N_PAD = 51200
ROWS_W = N_PAD // NW
ROWS_S = N_PAD // NSUB
E_PAD = 819200
EPW = E_PAD // NW
CHUNK = 1024
NCHUNK = EPW // CHUNK
NGRP = CHUNK // 16
ZROWS = 800

_mesh = plsc.VectorSubcoreMesh(
    core_axis_name="c", subcore_axis_name="s", num_cores=NC, num_subcores=NSUB
)


def _iota16():
    return lax.iota(jnp.int32, 16)


def _cfull(q):
    return jnp.full((16,), q, jnp.int32)


def _sigmoid(x):
    return 1.0 / (1.0 + jnp.exp(-x))


@functools.partial(
    pl.kernel,
    out_type=jax.ShapeDtypeStruct((N_PAD, 16), jnp.float32),
    mesh=_mesh,
    compiler_params=pltpu.CompilerParams(needs_layout_passes=False, use_tc_tiling_on_sc=False),
    scratch_types=[
        pltpu.VMEM((ROWS_W * 4,), jnp.float32),
        pltpu.VMEM((ROWS_W, 16), jnp.float32),
        pltpu.VMEM((8, 16), jnp.float32),
    ],
)
def _input_proj(xflat, w0, feat, xbuf, fbuf, wbuf):
    wid = lax.axis_index("c") * NSUB + lax.axis_index("s")
    base = wid * ROWS_W
    pltpu.sync_copy(xflat.at[pl.ds(base * 4, ROWS_W * 4)], xbuf)
    pltpu.sync_copy(w0, wbuf)
    iota = _iota16()

    def body(g, _):
        nid = iota + g * 16
        nid4 = nid * 4
        xs = [plsc.load_gather(xbuf, [nid4 + i]) for i in range(4)]
        for k in range(4):
            plsc.store_scatter(fbuf, [nid, _cfull(k)], xs[0] * wbuf[k])
        for c in range(3):
            for k in range(4):
                plsc.store_scatter(
                    fbuf, [nid, _cfull(4 + 4 * c + k)], xs[1 + c] * wbuf[4 + k]
                )
        return 0

    lax.fori_loop(0, ROWS_W // 16, body, 0)
    pltpu.sync_copy(fbuf, feat.at[pl.ds(base, ROWS_W)])


@functools.partial(
    pl.kernel,
    out_type=jax.ShapeDtypeStruct((NC, N_PAD, 16), jnp.float32),
    mesh=_mesh,
    compiler_params=pltpu.CompilerParams(needs_layout_passes=False, use_tc_tiling_on_sc=False),
    scratch_types=[
        pltpu.VMEM((8, 128), jnp.int32),
        pltpu.VMEM((8, 128), jnp.int32),
        pltpu.VMEM((CHUNK, 16), jnp.float32),
        pltpu.VMEM((CHUNK, 16), jnp.float32),
        pltpu.VMEM((CHUNK, 16), jnp.float32),
        pltpu.VMEM((256, 16), jnp.float32),
        pltpu.VMEM((ZROWS, 16), jnp.float32),
        pltpu.VMEM_SHARED((N_PAD, 16), jnp.float32),
        pltpu.SemaphoreType.DMA,
    ],
)
def _layer(feat, row2, col2, wv, part, ridx, cidx, b1, b2, mbuf, wbuf, zbuf, agg, sem):
    c = lax.axis_index("c")
    s = lax.axis_index("s")
    wid = c * NSUB + s

    def zb(i, _):
        zbuf[i] = jnp.zeros((16,), jnp.float32)
        return 0

    lax.fori_loop(0, ZROWS, zb, 0)
    for j in range(ROWS_S // ZROWS):
        pltpu.sync_copy(zbuf, agg.at[pl.ds(s * ROWS_S + j * ZROWS, ZROWS)])
    pltpu.sync_copy(wv, wbuf)
    plsc.subcore_barrier()

    iota = _iota16()

    def grp(g, _):
        eidx = iota + g * 16
        s1 = [plsc.load_gather(b1, [eidx, _cfull(k)]) for k in range(4)]
        s2 = [plsc.load_gather(b2, [eidx, _cfull(k)]) for k in range(4)]
        v1 = [
            [plsc.load_gather(b1, [eidx, _cfull(4 + 4 * c_ + k)]) for k in range(4)]
            for c_ in range(3)
        ]
        v2 = [
            [plsc.load_gather(b2, [eidx, _cfull(4 + 4 * c_ + k)]) for k in range(4)]
            for c_ in range(3)
        ]
        t = {}
        dot = {}
        for i in range(4):
            for j in range(4):
                t[i, j] = s1[i] * s2[j]
                dot[i, j] = (
                    v1[0][i] * v2[0][j] + v1[1][i] * v2[1][j] + v1[2][i] * v2[2][j]
                )
        for k in range(4):
            acc = t[0, 0] * wbuf[k]
            for i in range(4):
                for j in range(4):
                    if i == 0 and j == 0:
                        continue
                    acc = acc + t[i, j] * wbuf[i * 16 + j * 4 + k]
            for i in range(4):
                for j in range(4):
                    acc = acc + dot[i, j] * wbuf[64 + i * 16 + j * 4 + k]
            plsc.store_scatter(mbuf, [eidx, _cfull(k)], acc)
        u = [[None] * 4 for _ in range(4)]
        w2 = [[None] * 4 for _ in range(4)]
        for a in range(4):
            for k in range(4):
                u[a][k] = s1[0] * wbuf[128 + a * 4 + k]
                w2[a][k] = s2[0] * wbuf[192 + a * 16 + k]
                for b in range(1, 4):
                    u[a][k] = u[a][k] + s1[b] * wbuf[128 + b * 16 + a * 4 + k]
                    w2[a][k] = w2[a][k] + s2[b] * wbuf[192 + a * 16 + b * 4 + k]
        for c_ in range(3):
            for k in range(4):
                acc = u[0][k] * v2[c_][0] + w2[0][k] * v1[c_][0]
                for a in range(1, 4):
                    acc = acc + u[a][k] * v2[c_][a] + w2[a][k] * v1[c_][a]
                plsc.store_scatter(mbuf, [eidx, _cfull(4 + 4 * c_ + k)], acc)
        return 0

    def chunk_body(ch, _):
        r0 = wid * (EPW // 128) + ch * (CHUNK // 128)
        pltpu.sync_copy(row2.at[pl.ds(r0, 8)], ridx)
        pltpu.sync_copy(col2.at[pl.ds(r0, 8)], cidx)
        descs = []
        for j in range(8):
            descs.append(
                pltpu.async_copy(feat.at[ridx.at[j]], b1.at[pl.ds(j * 128, 128)], sem)
            )
            descs.append(
                pltpu.async_copy(feat.at[cidx.at[j]], b2.at[pl.ds(j * 128, 128)], sem)
            )
        for d in descs:
            d.wait()
        lax.fori_loop(0, NGRP, grp, 0)
        for j in range(8):
            pltpu.sync_copy(
                mbuf.at[pl.ds(j * 128, 128)], agg.at[ridx.at[j]], add=True
            )
        return 0

    lax.fori_loop(0, NCHUNK, chunk_body, 0)
    plsc.subcore_barrier()
    pltpu.sync_copy(
        agg.at[pl.ds(s * ROWS_S, ROWS_S)], part.at[c, pl.ds(s * ROWS_S, ROWS_S)]
    )


@functools.partial(
    pl.kernel,
    out_type=jax.ShapeDtypeStruct((N_PAD, 16), jnp.float32),
    mesh=_mesh,
    compiler_params=pltpu.CompilerParams(needs_layout_passes=False, use_tc_tiling_on_sc=False),
    scratch_types=[
        pltpu.VMEM((ROWS_W, 16), jnp.float32),
        pltpu.VMEM((ROWS_W, 16), jnp.float32),
    ],
)
def _gate(part, featn, a0, a1):
    wid = lax.axis_index("c") * NSUB + lax.axis_index("s")
    base = wid * ROWS_W
    pltpu.sync_copy(part.at[0, pl.ds(base, ROWS_W)], a0)
    pltpu.sync_copy(part.at[1, pl.ds(base, ROWS_W)], a1)
    rep = jnp.remainder(_iota16(), _cfull(4))

    def body(r, _):
        a = a0[r] + a1[r]
        a0[r] = a
        sg = plsc.load_gather(a0, [jnp.broadcast_to(r, (16,)), rep])
        a0[r] = a * _sigmoid(sg)
        return 0

    lax.fori_loop(0, ROWS_W, body, 0)
    pltpu.sync_copy(a0, featn.at[pl.ds(base, ROWS_W)])


@functools.partial(
    pl.kernel,
    out_type=jax.ShapeDtypeStruct((NC, 64), jnp.float32),
    mesh=_mesh,
    compiler_params=pltpu.CompilerParams(needs_layout_passes=False, use_tc_tiling_on_sc=False),
    scratch_types=[
        pltpu.VMEM((ROWS_W, 16), jnp.float32),
        pltpu.VMEM((ROWS_W, 16), jnp.float32),
        pltpu.VMEM((ROWS_W,), jnp.int32),
        pltpu.VMEM((4, 16), jnp.float32),
        pltpu.VMEM((128,), jnp.float32),
        pltpu.VMEM((NSUB, 128), jnp.float32),
        pltpu.VMEM((128,), jnp.float32),
        pltpu.VMEM_SHARED((NSUB, 128), jnp.float32),
    ],
)
def _readout(part, batchp, wout, en, a0, a1, bbuf, wbuf, evmem, ebuf, rbuf, eslots):
    c = lax.axis_index("c")
    s = lax.axis_index("s")
    wid = c * NSUB + s
    base = wid * ROWS_W
    pltpu.sync_copy(part.at[0, pl.ds(base, ROWS_W)], a0)
    pltpu.sync_copy(part.at[1, pl.ds(base, ROWS_W)], a1)
    pltpu.sync_copy(batchp.at[pl.ds(base, ROWS_W)], bbuf)
    pltpu.sync_copy(wout, wbuf)
    for j in range(8):
        evmem[pl.ds(j * 16, 16)] = jnp.zeros((16,), jnp.float32)
    iota = _iota16()

    def body(g, _):
        nid = iota + g * 16
        out = None
        for k in range(4):
            sk = plsc.load_gather(a0, [nid, _cfull(k)]) + plsc.load_gather(
                a1, [nid, _cfull(k)]
            )
            silu = sk * _sigmoid(sk)
            out = silu * wbuf[k] if out is None else out + silu * wbuf[k]
        bid = bbuf[pl.ds(g * 16, 16)]
        plsc.addupdate_scatter(evmem, [bid], out)
        return 0

    lax.fori_loop(0, ROWS_W // 16, body, 0)
    pltpu.sync_copy(evmem, eslots.at[s])
    plsc.subcore_barrier()

    @pl.when(s == 0)
    def _():
        pltpu.sync_copy(eslots, ebuf)
        for j in range(8):
            acc = ebuf[0, pl.ds(j * 16, 16)]
            for i in range(1, NSUB):
                acc = acc + ebuf[i, pl.ds(j * 16, 16)]
            rbuf[pl.ds(j * 16, 16)] = acc
        pltpu.sync_copy(rbuf.at[pl.ds(0, 64)], en.at[c])


def kernel(x, edge_index, batch, W_in_s, W_in_v, W_ss, W_vv, W_sv, W_vs, W_out):
    f32 = jnp.float32
    xp = jnp.zeros((N_PAD, 4), f32).at[:NN].set(x)
    row = edge_index[0].astype(jnp.int32)
    col = edge_index[1].astype(jnp.int32)
    rowp = jnp.concatenate([row, jnp.full((E_PAD - EE,), NN, jnp.int32)]).reshape(
        E_PAD // 128, 128
    )
    colp = jnp.concatenate([col, jnp.zeros((E_PAD - EE,), jnp.int32)]).reshape(
        E_PAD // 128, 128
    )
    batchp = jnp.concatenate(
        [batch.astype(jnp.int32), jnp.full((N_PAD - NN,), NB, jnp.int32)]
    )
    w0 = jnp.broadcast_to(
        jnp.concatenate([W_in_s[0], W_in_v[0]]).astype(f32)[:, None], (8, 16)
    )
    wvs = [
        jnp.broadcast_to(
            jnp.concatenate(
                [
                    W_ss[l].reshape(64),
                    W_vv[l].reshape(64),
                    W_sv[l].reshape(64),
                    W_vs[l].reshape(64),
                ]
            ).astype(f32)[:, None],
            (256, 16),
        )
        for l in range(3)
    ]
    wout = jnp.broadcast_to(W_out[:, 0].astype(f32)[:, None], (4, 16))

    feat = _input_proj(xp.reshape(-1), w0)
    part = None
    for l in range(3):
        part = _layer(feat, rowp, colp, wvs[l])
        if l < 2:
            feat = _gate(part)
    en = _readout(part, batchp, wout)
    return en[0] + en[1]


# speedup vs baseline: 82.1276x; 1.1255x over previous
import functools

import jax
import jax.numpy as jnp
from jax import lax
from jax.experimental import pallas as pl
from jax.experimental.pallas import tpu as pltpu
from jax.experimental.pallas import tpu_sc as plsc

NN = 50000
EE = 800000
NB = 64
NC = 2
NSUB = 16
NW = NC * NSUB

N_PAD = 51200
ROWS_W = N_PAD // NW
ROWS_S = N_PAD // NSUB
E_PAD = 819200
EPW = E_PAD // NW
CHUNK = 512
NCHUNK = EPW // CHUNK
NGRP = CHUNK // 16

_mesh = plsc.VectorSubcoreMesh(
    core_axis_name="c", subcore_axis_name="s", num_cores=NC, num_subcores=NSUB
)


def _iota16():
    return lax.iota(jnp.int32, 16)


def _cfull(q):
    return jnp.full((16,), q, jnp.int32)


def _sigmoid(x):
    return 1.0 / (1.0 + jnp.exp(-x))


def _bf16r(x):
    u = plsc.bitcast(x, jnp.uint32)
    odd = jnp.bitwise_and(jnp.right_shift(u, jnp.uint32(16)), jnp.uint32(1))
    r = jnp.bitwise_and(u + odd + jnp.uint32(0x7FFF), jnp.uint32(0xFFFF0000))
    return plsc.bitcast(r, jnp.float32)


@functools.partial(
    pl.kernel,
    out_type=jax.ShapeDtypeStruct((N_PAD, 16), jnp.float32),
    mesh=_mesh,
    compiler_params=pltpu.CompilerParams(needs_layout_passes=False, use_tc_tiling_on_sc=False),
    scratch_types=[
        pltpu.VMEM((ROWS_W * 4,), jnp.float32),
        pltpu.VMEM((ROWS_W, 16), jnp.float32),
        pltpu.VMEM((8, 16), jnp.float32),
    ],
)
def _input_proj(xflat, w0, feat, xbuf, fbuf, wbuf):
    wid = lax.axis_index("c") * NSUB + lax.axis_index("s")
    base = wid * ROWS_W
    pltpu.sync_copy(xflat.at[pl.ds(base * 4, ROWS_W * 4)], xbuf)
    pltpu.sync_copy(w0, wbuf)
    iota = _iota16()

    def body(g, _):
        nid = iota + g * 16
        nid4 = nid * 4
        xs = [plsc.load_gather(xbuf, [nid4 + i]) for i in range(4)]
        for k in range(4):
            plsc.store_scatter(fbuf, [nid, _cfull(k)], xs[0] * wbuf[k])
        for c in range(3):
            for k in range(4):
                plsc.store_scatter(
                    fbuf,
                    [nid, _cfull(4 + 4 * c + k)],
                    _bf16r(xs[1 + c] * wbuf[4 + k]),
                )
        return 0

    lax.fori_loop(0, ROWS_W // 16, body, 0)
    pltpu.sync_copy(fbuf, feat.at[pl.ds(base, ROWS_W)])


@functools.partial(
    pl.kernel,
    out_type=jax.ShapeDtypeStruct((NC, N_PAD, 16), jnp.float32),
    mesh=_mesh,
    compiler_params=pltpu.CompilerParams(needs_layout_passes=False, use_tc_tiling_on_sc=False),
    scratch_types=[
        pltpu.VMEM((4, 128), jnp.int32),
        pltpu.VMEM((4, 128), jnp.int32),
        pltpu.VMEM((4, 128), jnp.int32),
        pltpu.VMEM((4, 128), jnp.int32),
        pltpu.VMEM((CHUNK, 16), jnp.float32),
        pltpu.VMEM((CHUNK, 16), jnp.float32),
        pltpu.VMEM((CHUNK, 16), jnp.float32),
        pltpu.VMEM((CHUNK, 16), jnp.float32),
        pltpu.VMEM((CHUNK, 16), jnp.float32),
        pltpu.VMEM((256, 16), jnp.float32),
        pltpu.VMEM_SHARED((N_PAD, 16), jnp.float32),
        pltpu.SemaphoreType.DMA,
        pltpu.SemaphoreType.DMA,
    ],
)
def _layer(
    feat, row2, col2, wv, part,
    ridxA, cidxA, ridxB, cidxB, b1A, b2A, b1B, b2B, mbuf,
    wbuf, agg, semA, semB,
):
    c = lax.axis_index("c")
    s = lax.axis_index("s")
    wid = c * NSUB + s

    def zb(i, _):
        mbuf[i] = jnp.zeros((16,), jnp.float32)
        return 0

    lax.fori_loop(0, CHUNK, zb, 0)
    for j in range(ROWS_S // CHUNK):
        pltpu.sync_copy(mbuf, agg.at[pl.ds(s * ROWS_S + j * CHUNK, CHUNK)])
    pltpu.sync_copy(
        mbuf.at[pl.ds(0, ROWS_S % CHUNK)],
        agg.at[pl.ds(s * ROWS_S + (ROWS_S // CHUNK) * CHUNK, ROWS_S % CHUNK)],
    )
    pltpu.sync_copy(wv, wbuf)
    plsc.subcore_barrier()

    iota = _iota16()

    def stage_idx(ch, ridx, cidx):
        r0 = wid * (EPW // 128) + ch * (CHUNK // 128)
        pltpu.sync_copy(row2.at[pl.ds(r0, CHUNK // 128)], ridx)
        pltpu.sync_copy(col2.at[pl.ds(r0, CHUNK // 128)], cidx)

    def fire_gathers(ridx, cidx, b1, b2, sem):
        descs = []
        for j in range(CHUNK // 128):
            descs.append(
                pltpu.async_copy(feat.at[ridx.at[j]], b1.at[pl.ds(j * 128, 128)], sem)
            )
            descs.append(
                pltpu.async_copy(feat.at[cidx.at[j]], b2.at[pl.ds(j * 128, 128)], sem)
            )
        return descs

    def drain_gathers(ridx, cidx, b1, b2, sem):
        for j in range(CHUNK // 128):
            pltpu.make_async_copy(
                feat.at[ridx.at[j]], b1.at[pl.ds(j * 128, 128)], sem
            ).wait()
            pltpu.make_async_copy(
                feat.at[cidx.at[j]], b2.at[pl.ds(j * 128, 128)], sem
            ).wait()

    def make_grp(b1, b2):
        def grp(g, _):
            eidx = iota + g * 16
            s1 = [plsc.load_gather(b1, [eidx, _cfull(k)]) for k in range(4)]
            s2 = [plsc.load_gather(b2, [eidx, _cfull(k)]) for k in range(4)]
            v1 = [
                [plsc.load_gather(b1, [eidx, _cfull(4 + 4 * c_ + k)]) for k in range(4)]
                for c_ in range(3)
            ]
            v2 = [
                [plsc.load_gather(b2, [eidx, _cfull(4 + 4 * c_ + k)]) for k in range(4)]
                for c_ in range(3)
            ]
            t = {}
            dot = {}
            for i in range(4):
                for j in range(4):
                    t[i, j] = _bf16r(s1[i] * s2[j])
                    dot[i, j] = _bf16r(
                        v1[0][i] * v2[0][j] + v1[1][i] * v2[1][j] + v1[2][i] * v2[2][j]
                    )
            for k in range(4):
                acc = t[0, 0] * wbuf[k]
                for i in range(4):
                    for j in range(4):
                        if i == 0 and j == 0:
                            continue
                        acc = acc + t[i, j] * wbuf[i * 16 + j * 4 + k]
                acc2 = dot[0, 0] * wbuf[64 + k]
                for i in range(4):
                    for j in range(4):
                        if i == 0 and j == 0:
                            continue
                        acc2 = acc2 + dot[i, j] * wbuf[64 + i * 16 + j * 4 + k]
                plsc.store_scatter(mbuf, [eidx, _cfull(k)], acc + acc2)
            s1b = [_bf16r(s1[i]) for i in range(4)]
            s2b = [_bf16r(s2[j]) for j in range(4)]
            u = [[None] * 4 for _ in range(4)]
            w2 = [[None] * 4 for _ in range(4)]
            for a in range(4):
                for k in range(4):
                    u[a][k] = s1b[0] * wbuf[128 + a * 4 + k]
                    w2[a][k] = s2b[0] * wbuf[192 + a * 16 + k]
                    for b in range(1, 4):
                        u[a][k] = u[a][k] + s1b[b] * wbuf[128 + b * 16 + a * 4 + k]
                        w2[a][k] = w2[a][k] + s2b[b] * wbuf[192 + a * 16 + b * 4 + k]
                    u[a][k] = _bf16r(u[a][k])
                    w2[a][k] = _bf16r(w2[a][k])
            for c_ in range(3):
                for k in range(4):
                    p1 = u[0][k] * v2[c_][0]
                    p2 = w2[0][k] * v1[c_][0]
                    for a in range(1, 4):
                        p1 = p1 + u[a][k] * v2[c_][a]
                        p2 = p2 + w2[a][k] * v1[c_][a]
                    plsc.store_scatter(
                        mbuf, [eidx, _cfull(4 + 4 * c_ + k)], p1 + p2
                    )
            return 0

        return grp

    grpA = make_grp(b1A, b2A)
    grpB = make_grp(b1B, b2B)

    def scatter(ridx):
        for j in range(CHUNK // 128):
            pltpu.sync_copy(
                mbuf.at[pl.ds(j * 128, 128)], agg.at[ridx.at[j]], add=True
            )

    def chunk_body(ch, _):
        stage_idx(ch, ridxA, cidxA)
        fire_gathers(ridxA, cidxA, b1A, b2A, semA)
        drain_gathers(ridxA, cidxA, b1A, b2A, semA)
        lax.fori_loop(0, NGRP, grpA, 0)
        scatter(ridxA)
        return 0

    lax.fori_loop(0, NCHUNK, chunk_body, 0)

    plsc.subcore_barrier()
    pltpu.sync_copy(
        agg.at[pl.ds(s * ROWS_S, ROWS_S)], part.at[c, pl.ds(s * ROWS_S, ROWS_S)]
    )


@functools.partial(
    pl.kernel,
    out_type=jax.ShapeDtypeStruct((N_PAD, 16), jnp.float32),
    mesh=_mesh,
    compiler_params=pltpu.CompilerParams(needs_layout_passes=False, use_tc_tiling_on_sc=False),
    scratch_types=[
        pltpu.VMEM((ROWS_W, 16), jnp.float32),
        pltpu.VMEM((ROWS_W, 16), jnp.float32),
    ],
)
def _gate(part, featn, a0, a1):
    wid = lax.axis_index("c") * NSUB + lax.axis_index("s")
    base = wid * ROWS_W
    pltpu.sync_copy(part.at[0, pl.ds(base, ROWS_W)], a0)
    pltpu.sync_copy(part.at[1, pl.ds(base, ROWS_W)], a1)
    rep = jnp.remainder(_iota16(), _cfull(4))

    vmask = _iota16() < 4

    def body(r, _):
        a = a0[r] + a1[r]
        a0[r] = a
        sg = plsc.load_gather(a0, [jnp.broadcast_to(r, (16,)), rep])
        out = a * _sigmoid(sg)
        a0[r] = jnp.where(vmask, out, _bf16r(out))
        return 0

    lax.fori_loop(0, ROWS_W, body, 0)
    pltpu.sync_copy(a0, featn.at[pl.ds(base, ROWS_W)])


@functools.partial(
    pl.kernel,
    out_type=jax.ShapeDtypeStruct((NC, 64), jnp.float32),
    mesh=_mesh,
    compiler_params=pltpu.CompilerParams(needs_layout_passes=False, use_tc_tiling_on_sc=False),
    scratch_types=[
        pltpu.VMEM((ROWS_W, 16), jnp.float32),
        pltpu.VMEM((ROWS_W, 16), jnp.float32),
        pltpu.VMEM((ROWS_W,), jnp.int32),
        pltpu.VMEM((4, 16), jnp.float32),
        pltpu.VMEM((128,), jnp.float32),
        pltpu.VMEM((NSUB, 128), jnp.float32),
        pltpu.VMEM((128,), jnp.float32),
        pltpu.VMEM_SHARED((NSUB, 128), jnp.float32),
    ],
)
def _readout(part, batchp, wout, en, a0, a1, bbuf, wbuf, evmem, ebuf, rbuf, eslots):
    c = lax.axis_index("c")
    s = lax.axis_index("s")
    wid = c * NSUB + s
    base = wid * ROWS_W
    pltpu.sync_copy(part.at[0, pl.ds(base, ROWS_W)], a0)
    pltpu.sync_copy(part.at[1, pl.ds(base, ROWS_W)], a1)
    pltpu.sync_copy(batchp.at[pl.ds(base, ROWS_W)], bbuf)
    pltpu.sync_copy(wout, wbuf)
    for j in range(8):
        evmem[pl.ds(j * 16, 16)] = jnp.zeros((16,), jnp.float32)
    iota = _iota16()

    def body(g, _):
        nid = iota + g * 16
        out = None
        for k in range(4):
            sk = plsc.load_gather(a0, [nid, _cfull(k)]) + plsc.load_gather(
                a1, [nid, _cfull(k)]
            )
            silu = _bf16r(sk * _sigmoid(sk))
            out = silu * wbuf[k] if out is None else out + silu * wbuf[k]
        bid = bbuf[pl.ds(g * 16, 16)]
        plsc.addupdate_scatter(evmem, [bid], out)
        return 0

    lax.fori_loop(0, ROWS_W // 16, body, 0)
    pltpu.sync_copy(evmem, eslots.at[s])
    plsc.subcore_barrier()

    @pl.when(s == 0)
    def _():
        pltpu.sync_copy(eslots, ebuf)
        for j in range(8):
            acc = ebuf[0, pl.ds(j * 16, 16)]
            for i in range(1, NSUB):
                acc = acc + ebuf[i, pl.ds(j * 16, 16)]
            rbuf[pl.ds(j * 16, 16)] = acc
        pltpu.sync_copy(rbuf.at[pl.ds(0, 64)], en.at[c])


def kernel(x, edge_index, batch, W_in_s, W_in_v, W_ss, W_vv, W_sv, W_vs, W_out):
    f32 = jnp.float32
    xp = jnp.zeros((N_PAD, 4), f32).at[:NN].set(x)
    row = edge_index[0].astype(jnp.int32)
    col = edge_index[1].astype(jnp.int32)
    rowp = jnp.concatenate([row, jnp.full((E_PAD - EE,), NN, jnp.int32)]).reshape(
        E_PAD // 128, 128
    )
    colp = jnp.concatenate([col, jnp.zeros((E_PAD - EE,), jnp.int32)]).reshape(
        E_PAD // 128, 128
    )
    batchp = jnp.concatenate(
        [batch.astype(jnp.int32), jnp.full((N_PAD - NN,), NB, jnp.int32)]
    )
    w0 = jnp.broadcast_to(
        jnp.concatenate([W_in_s[0], W_in_v[0]]).astype(f32)[:, None], (8, 16)
    )
    def _wr(w):
        return w.astype(jnp.bfloat16).astype(f32)

    wvs = [
        jnp.broadcast_to(
            jnp.concatenate(
                [
                    _wr(W_ss[l]).reshape(64),
                    _wr(W_vv[l]).reshape(64),
                    _wr(W_sv[l]).reshape(64),
                    _wr(W_vs[l]).reshape(64),
                ]
            ).astype(f32)[:, None],
            (256, 16),
        )
        for l in range(3)
    ]
    wout = jnp.broadcast_to(
        W_out[:, 0].astype(jnp.bfloat16).astype(f32)[:, None], (4, 16)
    )

    feat = _input_proj(xp.reshape(-1), w0)
    part = None
    for l in range(3):
        part = _layer(feat, rowp, colp, wvs[l])
        if l < 2:
            feat = _gate(part)
    en = _readout(part, batchp, wout)
    return en[0] + en[1]
